# Initial kernel scaffold; baseline (speedup 1.0000x reference)
import jax
import jax.numpy as jnp
from jax.experimental import pallas as pl


def kernel(data, params, channel_transform, spatio_transform):
    raise NotImplementedError("write your pallas kernel here")


# speedup vs baseline: 17.9436x; 17.9436x over previous
import functools

import jax
import jax.numpy as jnp
from jax.experimental import pallas as pl
from jax.experimental.pallas import tpu as pltpu

_P = 5
_BR = 512
_BC = 2048


def _minmax_kernel(ct_ref, x_ref, mn_ref, mx_ref):
    i = pl.program_id(0)
    d = x_ref[...] * ct_ref[0, 0]
    bmn = jnp.min(d)
    bmx = jnp.max(d)

    @pl.when(i == 0)
    def _():
        mn_ref[0, 0] = bmn
        mx_ref[0, 0] = bmx

    @pl.when(i > 0)
    def _():
        mn_ref[0, 0] = jnp.minimum(mn_ref[0, 0], bmn)
        mx_ref[0, 0] = jnp.maximum(mx_ref[0, 0], bmx)


def _hist_kernel(ct_ref, mn_ref, mx_ref, x_ref, s_ref):
    i = pl.program_id(0)
    dmin = mn_ref[0, 0] - 0.1
    dmax = mx_ref[0, 0] + 0.1
    step = (dmax - dmin) / _P
    d = x_ref[...] * ct_ref[0, 0]

    @pl.when(i == 0)
    def _():
        for k in range(_P):
            s_ref[0, k] = 0

    for k in range(1, _P + 1):
        e = dmin + k * step
        s_ref[0, k - 1] += jnp.sum((d >= e).astype(jnp.int32))


def _map_kernel(nelem, ct_ref, st_ref, mn_ref, mx_ref, s_ref, prm_ref,
                x_ref, o_ref):
    f32 = jnp.float32
    ct = ct_ref[0, 0]
    st = st_ref[0, 0]
    dmin = mn_ref[0, 0] - 0.1
    dmax = mx_ref[0, 0] + 0.1
    step = (dmax - dmin) / _P
    e = [dmin + k * step for k in range(_P + 1)]

    S = [f32(nelem)] + [s_ref[0, k].astype(f32) for k in range(_P)]
    c = [S[k] - S[k + 1] for k in range(_P)]
    total = c[0] + c[1] + c[2] + c[3] + c[4]
    acc = []
    run = f32(0.0)
    for k in range(_P):
        run = run + c[k] / total
        acc.append(run * _P)
    grid = [(e[k] + e[k + 1]) * 0.5 for k in range(_P)]

    sl = [(acc[i + 1] - acc[i]) / (grid[i + 1] - grid[i]) for i in range(_P - 1)]
    a = [acc[i] - grid[i] * sl[i] for i in range(_P - 1)]

    def sel4(idx, v):
        return jnp.where(idx == 0, v[0],
               jnp.where(idx == 1, v[1],
               jnp.where(idx == 2, v[2], v[3])))

    frame = []
    for j in range(_P):
        jf = f32(j)
        t = jnp.int32(0)
        for k in range(_P):
            t = t + jnp.where(acc[k] < jf, 1, 0)
        ij = jnp.clip(t - 1, 0, _P - 2)
        x0 = sel4(ij, acc[0:4])
        x1 = sel4(ij, acc[1:5])
        y0 = sel4(ij, grid[0:4])
        y1 = sel4(ij, grid[1:5])
        frame.append(y0 + (jf - x0) / (x1 - x0) * (y1 - y0))

    pt = [frame[k] + 0.001 * prm_ref[0, k] for k in range(_P)]
    pv = [frame[k] + 0.001 * prm_ref[1, k] for k in range(_P)]

    d = x_ref[...] * ct
    i1 = (d > grid[1]).astype(f32)
    i2 = (d > grid[2]).astype(f32)
    i3 = (d > grid[3]).astype(f32)
    A = a[0] + i1 * (a[1] - a[0]) + i2 * (a[2] - a[1]) + i3 * (a[3] - a[2])
    B = sl[0] + i1 * (sl[1] - sl[0]) + i2 * (sl[2] - sl[1]) + i3 * (sl[3] - sl[2])
    index = A + B * d

    pos = index - jnp.floor(index)
    u1 = index >= 1.0
    u2 = index >= 2.0
    u3 = index >= 3.0
    u4 = index >= 4.0

    def pick(tbl, lo):
        r = tbl[lo]
        for k, u in zip(range(lo + 1, _P), (u1, u2, u3, u4)):
            r = jnp.where(u, tbl[k], r)
        return r

    thB = pick(pt, 0)
    thE = pick(pt, 1)
    theta = thB + pos * (thE - thB)

    veB = pick(pv, 0)
    veE = pick(pv, 1)
    velo = veB + pos * (veE - veB)

    dx = velo * jnp.cos(theta)
    dy = velo * jnp.sin(theta)
    o_ref[...] = (d * (1.0 + dy) + dx) * st


def _smem_spec():
    return pl.BlockSpec(memory_space=pltpu.SMEM)


@jax.jit
def kernel(data, params, channel_transform, spatio_transform):
    shape = data.shape
    cols = _BC
    rows = data.size // cols
    x = data.reshape(rows, cols)
    nblk = rows // _BR
    ct = channel_transform.reshape(1, 1)
    st = spatio_transform.reshape(1, 1)
    prm = params.reshape(2, _P)

    blk = pl.BlockSpec((_BR, cols), lambda i: (i, 0))

    mn, mx = pl.pallas_call(
        _minmax_kernel,
        grid=(nblk,),
        in_specs=[_smem_spec(), blk],
        out_specs=[_smem_spec(), _smem_spec()],
        out_shape=[jax.ShapeDtypeStruct((1, 1), jnp.float32),
                   jax.ShapeDtypeStruct((1, 1), jnp.float32)],
    )(ct, x)

    s = pl.pallas_call(
        _hist_kernel,
        grid=(nblk,),
        in_specs=[_smem_spec(), _smem_spec(), _smem_spec(), blk],
        out_specs=_smem_spec(),
        out_shape=jax.ShapeDtypeStruct((1, _P), jnp.int32),
    )(ct, mn, mx, x)

    out = pl.pallas_call(
        functools.partial(_map_kernel, float(rows * cols)),
        grid=(nblk,),
        in_specs=[_smem_spec()] * 6 + [blk],
        out_specs=blk,
        out_shape=jax.ShapeDtypeStruct((rows, cols), jnp.float32),
    )(ct, st, mn, mx, s, prm, x)

    return out.reshape(shape)


# speedup vs baseline: 20.0032x; 1.1148x over previous
import functools

import jax
import jax.numpy as jnp
from jax.experimental import pallas as pl
from jax.experimental.pallas import tpu as pltpu

_P = 5
_BR = 512
_BC = 2048


def _minmax_kernel(ct_ref, x_ref, mn_ref, mx_ref):
    i = pl.program_id(0)
    d = x_ref[...] * ct_ref[0, 0]
    bmn = jnp.min(d)
    bmx = jnp.max(d)

    @pl.when(i == 0)
    def _():
        mn_ref[0, 0] = bmn
        mx_ref[0, 0] = bmx

    @pl.when(i > 0)
    def _():
        mn_ref[0, 0] = jnp.minimum(mn_ref[0, 0], bmn)
        mx_ref[0, 0] = jnp.maximum(mx_ref[0, 0], bmx)


def _hist_kernel(ct_ref, mn_ref, mx_ref, x_ref, s_ref):
    i = pl.program_id(0)
    dmin = mn_ref[0, 0] - 0.1
    dmax = mx_ref[0, 0] + 0.1
    step = (dmax - dmin) / _P
    d = x_ref[...] * ct_ref[0, 0]

    @pl.when(i == 0)
    def _():
        for k in range(_P):
            s_ref[0, k] = 0

    for k in range(1, _P + 1):
        e = dmin + k * step
        s_ref[0, k - 1] += jnp.sum((d >= e).astype(jnp.int32))


def _sincos(t):
    f32 = jnp.float32
    q = t * f32(0.6366197723675814)
    nf = jnp.round(q)
    n = nf.astype(jnp.int32)
    f = t - nf * f32(1.570556640625)
    f = f - nf * f32(0.00023968616733327508)
    f2 = f * f
    ps = f32(-1.9515296e-4) + f2 * f32(2.7526994e-6)
    ps = f32(8.3321608e-3) + f2 * ps
    ps = f32(-1.6666654e-1) + f2 * ps
    s = f + f * f2 * ps
    pc = f32(-1.388731625e-3) + f2 * f32(2.443315711e-5)
    pc = f32(4.16666418e-2) + f2 * pc
    pc = f32(-0.5) + f2 * pc
    c = f32(1.0) + f2 * pc
    swap = (n & 1) != 0
    sb = jnp.where(swap, c, s)
    cb = jnp.where(swap, s, c)
    neg_s = (n & 2) != 0
    neg_c = jnp.logical_xor(swap, neg_s)
    sin_t = jnp.where(neg_s, -sb, sb)
    cos_t = jnp.where(neg_c, -cb, cb)
    return sin_t, cos_t


def _map_kernel(nelem, ct_ref, st_ref, mn_ref, mx_ref, s_ref, prm_ref,
                x_ref, o_ref):
    f32 = jnp.float32
    ct = ct_ref[0, 0]
    st = st_ref[0, 0]
    dmin = mn_ref[0, 0] - 0.1
    dmax = mx_ref[0, 0] + 0.1
    step = (dmax - dmin) / _P
    e = [dmin + k * step for k in range(_P + 1)]

    S = [f32(nelem)] + [s_ref[0, k].astype(f32) for k in range(_P)]
    c = [S[k] - S[k + 1] for k in range(_P)]
    total = c[0] + c[1] + c[2] + c[3] + c[4]
    acc = []
    run = f32(0.0)
    for k in range(_P):
        run = run + c[k] / total
        acc.append(run * _P)
    grid = [(e[k] + e[k + 1]) * 0.5 for k in range(_P)]

    sl = [(acc[i + 1] - acc[i]) / (grid[i + 1] - grid[i]) for i in range(_P - 1)]
    a = [acc[i] - grid[i] * sl[i] for i in range(_P - 1)]

    def sel4(idx, v):
        return jnp.where(idx == 0, v[0],
               jnp.where(idx == 1, v[1],
               jnp.where(idx == 2, v[2], v[3])))

    frame = []
    for j in range(_P):
        jf = f32(j)
        t = jnp.int32(0)
        for k in range(_P):
            t = t + jnp.where(acc[k] < jf, 1, 0)
        ij = jnp.clip(t - 1, 0, _P - 2)
        x0 = sel4(ij, acc[0:4])
        x1 = sel4(ij, acc[1:5])
        y0 = sel4(ij, grid[0:4])
        y1 = sel4(ij, grid[1:5])
        frame.append(y0 + (jf - x0) / (x1 - x0) * (y1 - y0))

    pt = [frame[k] + 0.001 * prm_ref[0, k] for k in range(_P)]
    pv = [frame[k] + 0.001 * prm_ref[1, k] for k in range(_P)]

    d = x_ref[...] * ct
    i1 = (d > grid[1]).astype(f32)
    i2 = (d > grid[2]).astype(f32)
    i3 = (d > grid[3]).astype(f32)
    A = a[0] + i1 * (a[1] - a[0]) + i2 * (a[2] - a[1]) + i3 * (a[3] - a[2])
    B = sl[0] + i1 * (sl[1] - sl[0]) + i2 * (sl[2] - sl[1]) + i3 * (sl[3] - sl[2])
    index = A + B * d

    pos = index - jnp.floor(index)
    u1 = index >= 1.0
    u2 = index >= 2.0
    u3 = index >= 3.0
    u4 = index >= 4.0

    def pick(tbl, lo):
        r = tbl[lo]
        for k, u in zip(range(lo + 1, _P), (u1, u2, u3, u4)):
            r = jnp.where(u, tbl[k], r)
        return r

    thB = pick(pt, 0)
    thE = pick(pt, 1)
    theta = thB + pos * (thE - thB)

    veB = pick(pv, 0)
    veE = pick(pv, 1)
    velo = veB + pos * (veE - veB)

    sin_t, cos_t = _sincos(theta)
    dx = velo * cos_t
    dy = velo * sin_t
    o_ref[...] = (d * (1.0 + dy) + dx) * st


def _smem_spec():
    return pl.BlockSpec(memory_space=pltpu.SMEM)


@jax.jit
def kernel(data, params, channel_transform, spatio_transform):
    shape = data.shape
    cols = _BC
    rows = data.size // cols
    x = data.reshape(rows, cols)
    nblk = rows // _BR
    ct = channel_transform.reshape(1, 1)
    st = spatio_transform.reshape(1, 1)
    prm = params.reshape(2, _P)

    blk = pl.BlockSpec((_BR, cols), lambda i: (i, 0))

    mn, mx = pl.pallas_call(
        _minmax_kernel,
        grid=(nblk,),
        in_specs=[_smem_spec(), blk],
        out_specs=[_smem_spec(), _smem_spec()],
        out_shape=[jax.ShapeDtypeStruct((1, 1), jnp.float32),
                   jax.ShapeDtypeStruct((1, 1), jnp.float32)],
    )(ct, x)

    s = pl.pallas_call(
        _hist_kernel,
        grid=(nblk,),
        in_specs=[_smem_spec(), _smem_spec(), _smem_spec(), blk],
        out_specs=_smem_spec(),
        out_shape=jax.ShapeDtypeStruct((1, _P), jnp.int32),
    )(ct, mn, mx, x)

    out = pl.pallas_call(
        functools.partial(_map_kernel, float(rows * cols)),
        grid=(nblk,),
        in_specs=[_smem_spec()] * 6 + [blk],
        out_specs=blk,
        out_shape=jax.ShapeDtypeStruct((rows, cols), jnp.float32),
    )(ct, st, mn, mx, s, prm, x)

    return out.reshape(shape)


# speedup vs baseline: 21.3910x; 1.0694x over previous
import functools

import jax
import jax.numpy as jnp
from jax.experimental import pallas as pl
from jax.experimental.pallas import tpu as pltpu

_P = 5
_BR = 512
_BC = 2048

_INV2PI = 0.15915494309189535
_SINC = (6.2831854820251465, -41.34170150756836, 81.60515594482422,
         -76.70345306396484, 42.02960205078125, -14.913920402526855,
         3.2582054138183594)
_COSC = (1.0, -19.739206314086914, 64.93917083740234, -85.45116424560547,
         60.176231384277344, -26.000532150268555, 6.57561731338501)


def _minmax_kernel(ct_ref, x_ref, mn_ref, mx_ref):
    i = pl.program_id(0)
    d = x_ref[...] * ct_ref[0, 0]
    bmn = jnp.min(d)
    bmx = jnp.max(d)

    @pl.when(i == 0)
    def _():
        mn_ref[0, 0] = bmn
        mx_ref[0, 0] = bmx

    @pl.when(i > 0)
    def _():
        mn_ref[0, 0] = jnp.minimum(mn_ref[0, 0], bmn)
        mx_ref[0, 0] = jnp.maximum(mx_ref[0, 0], bmx)


def _hist_kernel(ct_ref, mn_ref, mx_ref, x_ref, s_ref):
    i = pl.program_id(0)
    dmin = mn_ref[0, 0] - 0.1
    dmax = mx_ref[0, 0] + 0.1
    step = (dmax - dmin) / _P
    d = x_ref[...] * ct_ref[0, 0]

    @pl.when(i == 0)
    def _():
        for k in range(_P):
            s_ref[0, k] = 0

    for k in range(1, _P + 1):
        e = dmin + k * step
        s_ref[0, k - 1] += jnp.sum((d >= e).astype(jnp.int32))


def _finz(v):
    return jnp.where(jnp.abs(v) < jnp.inf, v, 0.0)


def _hinge3(x, a1, a2, a3, c, m, g1, g2, g3):
    return c + m * x + g1 * a1 + g2 * a2 + g3 * a3


def _pwl_coeffs(p, f32):
    s = [p[j + 1] - p[j] for j in range(4)]
    g1 = (s[1] - s[0]) * f32(0.5)
    g2 = (s[2] - s[1]) * f32(0.5)
    g3 = (s[3] - s[2]) * f32(0.5)
    m = s[0] + g1 + g2 + g3
    c = p[0] - (g1 + 2.0 * g2 + 3.0 * g3)
    return c, m, g1, g2, g3


def _map_kernel(nelem, ct_ref, st_ref, mn_ref, mx_ref, s_ref, prm_ref,
                x_ref, o_ref):
    f32 = jnp.float32
    ct = ct_ref[0, 0]
    st = st_ref[0, 0]
    dmin = mn_ref[0, 0] - 0.1
    dmax = mx_ref[0, 0] + 0.1
    step = (dmax - dmin) / _P
    e = [dmin + k * step for k in range(_P + 1)]

    S = [f32(nelem)] + [s_ref[0, k].astype(f32) for k in range(_P)]
    c = [S[k] - S[k + 1] for k in range(_P)]
    total = c[0] + c[1] + c[2] + c[3] + c[4]
    acc = []
    run = f32(0.0)
    for k in range(_P):
        run = run + c[k] / total
        acc.append(run * _P)
    grid = [(e[k] + e[k + 1]) * 0.5 for k in range(_P)]

    sl = [(acc[i + 1] - acc[i]) / (grid[i + 1] - grid[i]) for i in range(4)]
    gI1 = (sl[1] - sl[0]) * f32(0.5)
    gI2 = (sl[2] - sl[1]) * f32(0.5)
    gI3 = (sl[3] - sl[2]) * f32(0.5)
    mI = sl[0] + gI1 + gI2 + gI3
    cI = acc[1] - mI * grid[1] - gI2 * (grid[2] - grid[1]) \
        - gI3 * (grid[3] - grid[1])

    def sel4(idx, v):
        return jnp.where(idx == 0, v[0],
               jnp.where(idx == 1, v[1],
               jnp.where(idx == 2, v[2], v[3])))

    frame = []
    for j in range(_P):
        jf = f32(j)
        t = jnp.int32(0)
        for k in range(_P):
            t = t + jnp.where(acc[k] < jf, 1, 0)
        ij = jnp.clip(t - 1, 0, _P - 2)
        x0 = sel4(ij, acc[0:4])
        x1 = sel4(ij, acc[1:5])
        y0 = sel4(ij, grid[0:4])
        y1 = sel4(ij, grid[1:5])
        frame.append(y0 + (jf - x0) / (x1 - x0) * (y1 - y0))

    pt = [_finz(frame[k] + 0.001 * prm_ref[0, k]) for k in range(_P)]
    pv = [_finz(frame[k] + 0.001 * prm_ref[1, k]) for k in range(_P)]
    cT, mT, gT1, gT2, gT3 = _pwl_coeffs(pt, f32)
    cV, mV, gV1, gV2, gV3 = _pwl_coeffs(pv, f32)

    d = x_ref[...] * ct
    a1 = jnp.abs(d - grid[1])
    a2 = jnp.abs(d - grid[2])
    a3 = jnp.abs(d - grid[3])
    index = _hinge3(d, a1, a2, a3, cI, mI, gI1, gI2, gI3)

    frac = index - jnp.floor(index)
    z = jnp.where(index < 0.0, frac, jnp.minimum(index, f32(4.0)))
    b1 = jnp.abs(z - f32(1.0))
    b2 = jnp.abs(z - f32(2.0))
    b3 = jnp.abs(z - f32(3.0))
    theta = _hinge3(z, b1, b2, b3, cT, mT, gT1, gT2, gT3)
    velo = _hinge3(z, b1, b2, b3, cV, mV, gV1, gV2, gV3)

    r = theta * f32(_INV2PI)
    u = r - jnp.round(r)
    T = u * u
    ps = f32(_SINC[6])
    pc = f32(_COSC[6])
    for k in range(5, -1, -1):
        ps = f32(_SINC[k]) + T * ps
        pc = f32(_COSC[k]) + T * pc
    sin_t = u * ps
    cos_t = pc

    dy = velo * sin_t
    dx = velo * cos_t
    o_ref[...] = (d * (1.0 + dy) + dx) * st


def _smem_spec():
    return pl.BlockSpec(memory_space=pltpu.SMEM)


@jax.jit
def kernel(data, params, channel_transform, spatio_transform):
    shape = data.shape
    cols = _BC
    rows = data.size // cols
    x = data.reshape(rows, cols)
    nblk = rows // _BR
    ct = channel_transform.reshape(1, 1)
    st = spatio_transform.reshape(1, 1)
    prm = params.reshape(2, _P)

    blk = pl.BlockSpec((_BR, cols), lambda i: (i, 0))

    mn, mx = pl.pallas_call(
        _minmax_kernel,
        grid=(nblk,),
        in_specs=[_smem_spec(), blk],
        out_specs=[_smem_spec(), _smem_spec()],
        out_shape=[jax.ShapeDtypeStruct((1, 1), jnp.float32),
                   jax.ShapeDtypeStruct((1, 1), jnp.float32)],
    )(ct, x)

    s = pl.pallas_call(
        _hist_kernel,
        grid=(nblk,),
        in_specs=[_smem_spec(), _smem_spec(), _smem_spec(), blk],
        out_specs=_smem_spec(),
        out_shape=jax.ShapeDtypeStruct((1, _P), jnp.int32),
    )(ct, mn, mx, x)

    out = pl.pallas_call(
        functools.partial(_map_kernel, float(rows * cols)),
        grid=(nblk,),
        in_specs=[_smem_spec()] * 6 + [blk],
        out_specs=blk,
        out_shape=jax.ShapeDtypeStruct((rows, cols), jnp.float32),
    )(ct, st, mn, mx, s, prm, x)

    return out.reshape(shape)


# speedup vs baseline: 42.0460x; 1.9656x over previous
import functools

import jax
import jax.numpy as jnp
from jax import lax
from jax.experimental import pallas as pl
from jax.experimental.pallas import tpu as pltpu
from jax.experimental.pallas import tpu_sc as plsc

_P = 5
_BR = 512
_BC = 2048

_INV2PI = 0.15915494309189535
_SINC = (6.2831854820251465, -41.34170150756836, 81.60515594482422,
         -76.70345306396484, 42.02960205078125, -14.913920402526855,
         3.2582054138183594)
_COSC = (1.0, -19.739206314086914, 64.93917083740234, -85.45116424560547,
         60.176231384277344, -26.000532150268555, 6.57561731338501)


def _minmax_kernel(ct_ref, x_ref, mn_ref, mx_ref):
    i = pl.program_id(0)
    d = x_ref[...] * ct_ref[0, 0]
    bmn = jnp.min(d)
    bmx = jnp.max(d)

    @pl.when(i == 0)
    def _():
        mn_ref[0, 0] = bmn
        mx_ref[0, 0] = bmx

    @pl.when(i > 0)
    def _():
        mn_ref[0, 0] = jnp.minimum(mn_ref[0, 0], bmn)
        mx_ref[0, 0] = jnp.maximum(mx_ref[0, 0], bmx)


def _hist_kernel(ct_ref, mn_ref, mx_ref, x_ref, s_ref):
    i = pl.program_id(0)
    dmin = mn_ref[0, 0] - 0.1
    dmax = mx_ref[0, 0] + 0.1
    step = (dmax - dmin) / _P
    d = x_ref[...] * ct_ref[0, 0]

    @pl.when(i == 0)
    def _():
        for k in range(_P):
            s_ref[0, k] = 0

    for k in range(1, _P + 1):
        e = dmin + k * step
        s_ref[0, k - 1] += jnp.sum((d >= e).astype(jnp.int32))


def _finz(v):
    return jnp.where(jnp.abs(v) < jnp.inf, v, 0.0)


def _hinge3(x, a1, a2, a3, c, m, g1, g2, g3):
    return c + m * x + g1 * a1 + g2 * a2 + g3 * a3


def _pwl_coeffs(p, f32):
    s = [p[j + 1] - p[j] for j in range(4)]
    g1 = (s[1] - s[0]) * f32(0.5)
    g2 = (s[2] - s[1]) * f32(0.5)
    g3 = (s[3] - s[2]) * f32(0.5)
    m = s[0] + g1 + g2 + g3
    c = p[0] - (g1 + 2.0 * g2 + 3.0 * g3)
    return c, m, g1, g2, g3


def _map_kernel(nelem, ct_ref, st_ref, mn_ref, mx_ref, s_ref, prm_ref,
                x_ref, o_ref, pre_ref):
    f32 = jnp.float32

    @pl.when(pl.program_id(0) == 0)
    def _():
        _map_preamble(nelem, ct_ref, st_ref, mn_ref, mx_ref, s_ref, prm_ref,
                      pre_ref)

    v = [pre_ref[0, k] for k in range(20)]
    (ct, st, G1, G2, G3, cI, mI, gI1, gI2, gI3,
     cT, mT, gT1, gT2, gT3, cV, mV, gV1, gV2, gV3) = v

    d = x_ref[...] * ct
    a1 = jnp.abs(d - G1)
    a2 = jnp.abs(d - G2)
    a3 = jnp.abs(d - G3)
    index = _hinge3(d, a1, a2, a3, cI, mI, gI1, gI2, gI3)

    frac = index - jnp.floor(index)
    z = jnp.where(index < 0.0, frac, jnp.minimum(index, f32(4.0)))
    b1 = jnp.abs(z - f32(1.0))
    b2 = jnp.abs(z - f32(2.0))
    b3 = jnp.abs(z - f32(3.0))
    theta = _hinge3(z, b1, b2, b3, cT, mT, gT1, gT2, gT3)
    velo = _hinge3(z, b1, b2, b3, cV, mV, gV1, gV2, gV3)

    r = theta * f32(_INV2PI)
    u = r - jnp.round(r)
    T = u * u
    ps = f32(_SINC[6])
    pc = f32(_COSC[6])
    for k in range(5, -1, -1):
        ps = f32(_SINC[k]) + T * ps
        pc = f32(_COSC[k]) + T * pc
    sin_t = u * ps
    cos_t = pc

    dy = velo * sin_t
    dx = velo * cos_t
    o_ref[...] = (d * (1.0 + dy) + dx) * st


def _map_preamble(nelem, ct_ref, st_ref, mn_ref, mx_ref, s_ref, prm_ref,
                  pre_ref):
    f32 = jnp.float32
    ct = ct_ref[0, 0]
    st = st_ref[0, 0]
    dmin = mn_ref[0, 0] - 0.1
    dmax = mx_ref[0, 0] + 0.1
    step = (dmax - dmin) / _P
    e = [dmin + k * step for k in range(_P + 1)]

    S = [f32(nelem)] + [s_ref[0, k].astype(f32) for k in range(_P)]
    c = [S[k] - S[k + 1] for k in range(_P)]
    total = c[0] + c[1] + c[2] + c[3] + c[4]
    acc = []
    run = f32(0.0)
    for k in range(_P):
        run = run + c[k] / total
        acc.append(run * _P)
    grid = [(e[k] + e[k + 1]) * 0.5 for k in range(_P)]

    sl = [(acc[i + 1] - acc[i]) / (grid[i + 1] - grid[i]) for i in range(4)]
    gI1 = (sl[1] - sl[0]) * f32(0.5)
    gI2 = (sl[2] - sl[1]) * f32(0.5)
    gI3 = (sl[3] - sl[2]) * f32(0.5)
    mI = sl[0] + gI1 + gI2 + gI3
    cI = acc[1] - mI * grid[1] - gI2 * (grid[2] - grid[1]) \
        - gI3 * (grid[3] - grid[1])

    def sel4(idx, v):
        return jnp.where(idx == 0, v[0],
               jnp.where(idx == 1, v[1],
               jnp.where(idx == 2, v[2], v[3])))

    frame = []
    for j in range(_P):
        jf = f32(j)
        t = jnp.int32(0)
        for k in range(_P):
            t = t + jnp.where(acc[k] < jf, 1, 0)
        ij = jnp.clip(t - 1, 0, _P - 2)
        x0 = sel4(ij, acc[0:4])
        x1 = sel4(ij, acc[1:5])
        y0 = sel4(ij, grid[0:4])
        y1 = sel4(ij, grid[1:5])
        frame.append(y0 + (jf - x0) / (x1 - x0) * (y1 - y0))

    pt = [_finz(frame[k] + 0.001 * prm_ref[0, k]) for k in range(_P)]
    pv = [_finz(frame[k] + 0.001 * prm_ref[1, k]) for k in range(_P)]
    cT, mT, gT1, gT2, gT3 = _pwl_coeffs(pt, f32)
    cV, mV, gV1, gV2, gV3 = _pwl_coeffs(pv, f32)

    vals = (ct, st, grid[1], grid[2], grid[3], cI, mI, gI1, gI2, gI3,
            cT, mT, gT1, gT2, gT3, cV, mV, gV1, gV2, gV3)
    for k, val in enumerate(vals):
        pre_ref[0, k] = val


_NW = 32
_NELEM = 4 * 4096 * 2048
_PER_W = _NELEM // _NW
_SC_CH = 32768
_NCH = _PER_W // _SC_CH


def _sc_minmax_body(x_hbm, o_hbm, buf0, buf1, acc, sem0, sem1):
    wid = lax.axis_index("s") * 2 + lax.axis_index("c")
    base = wid * _PER_W
    bufs = (buf0, buf1)
    sems = (sem0, sem1)
    cps = [None, None]
    cps[0] = pltpu.async_copy(x_hbm.at[pl.ds(base, _SC_CH)], buf0, sem0)
    mn = jnp.full((16,), jnp.inf, jnp.float32)
    mx = jnp.full((16,), -jnp.inf, jnp.float32)
    for j in range(_NCH):
        if j + 1 < _NCH:
            nb = (j + 1) % 2
            cps[nb] = pltpu.async_copy(
                x_hbm.at[pl.ds(base + (j + 1) * _SC_CH, _SC_CH)],
                bufs[nb], sems[nb])
        cps[j % 2].wait()
        buf = bufs[j % 2]

        def body(i, carry, buf=buf):
            m, M = carry
            v = buf[pl.ds(i * 16, 16)]
            return jnp.minimum(m, v), jnp.maximum(M, v)

        mn, mx = lax.fori_loop(0, _SC_CH // 16, body, (mn, mx))
    acc[pl.ds(0, 16)] = mn
    acc[pl.ds(16, 16)] = mx
    pltpu.sync_copy(acc, o_hbm.at[wid])


def _sc_hist_body(x_hbm, ctv_hbm, e_hbm, o_hbm, buf0, buf1, ctv, ev, acc,
                  sem0, sem1):
    wid = lax.axis_index("s") * 2 + lax.axis_index("c")
    base = wid * _PER_W
    pltpu.sync_copy(ctv_hbm, ctv)
    pltpu.sync_copy(e_hbm, ev)
    ct = jnp.min(ctv[...])
    e = [jnp.min(ev[pl.ds(16 * k, 16)]) for k in range(_P)]
    bufs = (buf0, buf1)
    sems = (sem0, sem1)
    cps = [None, None]
    cps[0] = pltpu.async_copy(x_hbm.at[pl.ds(base, _SC_CH)], buf0, sem0)
    z = jnp.zeros((16,), jnp.int32)
    accs = (z, z, z, z, z)
    for j in range(_NCH):
        if j + 1 < _NCH:
            nb = (j + 1) % 2
            cps[nb] = pltpu.async_copy(
                x_hbm.at[pl.ds(base + (j + 1) * _SC_CH, _SC_CH)],
                bufs[nb], sems[nb])
        cps[j % 2].wait()
        buf = bufs[j % 2]

        def body(i, carry, buf=buf):
            d = buf[pl.ds(i * 16, 16)] * ct
            return tuple(
                cc + (d >= ek).astype(jnp.int32) for cc, ek in zip(carry, e))

        accs = lax.fori_loop(0, _SC_CH // 16, body, accs)
    iota = lax.iota(jnp.int32, 16)
    out = jnp.zeros((16,), jnp.int32)
    for k in range(_P):
        out = jnp.where(iota == k, jnp.sum(accs[k]), out)
    acc[...] = out
    pltpu.sync_copy(acc, o_hbm.at[wid])


_SC_MESH = dict(core_axis_name="c", subcore_axis_name="s")


def _sc_minmax(x):
    f = pl.kernel(
        _sc_minmax_body,
        mesh=plsc.VectorSubcoreMesh(**_SC_MESH),
        out_type=jax.ShapeDtypeStruct((_NW, 32), jnp.float32),
        scratch_types=[
            pltpu.VMEM((_SC_CH,), jnp.float32),
            pltpu.VMEM((_SC_CH,), jnp.float32),
            pltpu.VMEM((32,), jnp.float32),
            pltpu.SemaphoreType.DMA,
            pltpu.SemaphoreType.DMA,
        ],
    )
    return f(x)


def _sc_hist(x, ctv, ev):
    f = pl.kernel(
        _sc_hist_body,
        mesh=plsc.VectorSubcoreMesh(**_SC_MESH),
        out_type=jax.ShapeDtypeStruct((_NW, 16), jnp.int32),
        scratch_types=[
            pltpu.VMEM((_SC_CH,), jnp.float32),
            pltpu.VMEM((_SC_CH,), jnp.float32),
            pltpu.VMEM((16,), jnp.float32),
            pltpu.VMEM((16 * _P,), jnp.float32),
            pltpu.VMEM((16,), jnp.int32),
            pltpu.SemaphoreType.DMA,
            pltpu.SemaphoreType.DMA,
        ],
    )
    return f(x, ctv, ev)


def _smem_spec():
    return pl.BlockSpec(memory_space=pltpu.SMEM)


@jax.jit
def kernel(data, params, channel_transform, spatio_transform):
    shape = data.shape
    cols = _BC
    rows = data.size // cols
    x = data.reshape(rows, cols)
    nblk = rows // _BR
    ct = channel_transform.reshape(1, 1)
    st = spatio_transform.reshape(1, 1)
    prm = params.reshape(2, _P)

    blk = pl.BlockSpec((_BR, cols), lambda i: (i, 0))

    mn, mx = pl.pallas_call(
        _minmax_kernel,
        grid=(nblk,),
        in_specs=[_smem_spec(), blk],
        out_specs=[_smem_spec(), _smem_spec()],
        out_shape=[jax.ShapeDtypeStruct((1, 1), jnp.float32),
                   jax.ShapeDtypeStruct((1, 1), jnp.float32)],
    )(ct, x)

    s = pl.pallas_call(
        _hist_kernel,
        grid=(nblk,),
        in_specs=[_smem_spec(), _smem_spec(), _smem_spec(), blk],
        out_specs=_smem_spec(),
        out_shape=jax.ShapeDtypeStruct((1, _P), jnp.int32),
    )(ct, mn, mx, x)

    out = pl.pallas_call(
        functools.partial(_map_kernel, float(rows * cols)),
        grid=(nblk,),
        in_specs=[_smem_spec()] * 6 + [blk],
        out_specs=blk,
        out_shape=jax.ShapeDtypeStruct((rows, cols), jnp.float32),
        scratch_shapes=[pltpu.SMEM((1, 20), jnp.float32)],
    )(ct, st, mn, mx, s, prm, x)

    return out.reshape(shape)


# speedup vs baseline: 42.0476x; 1.0000x over previous
import functools

import jax
import jax.numpy as jnp
from jax.experimental import pallas as pl
from jax.experimental.pallas import tpu as pltpu

_P = 5
_BR = 512
_BC = 2048

_INV2PI = 0.15915494309189535
_SINC = (6.2831854820251465, -41.34170150756836, 81.60515594482422,
         -76.70345306396484, 42.02960205078125, -14.913920402526855,
         3.2582054138183594)
_COSC = (1.0, -19.739206314086914, 64.93917083740234, -85.45116424560547,
         60.176231384277344, -26.000532150268555, 6.57561731338501)


def _minmax_kernel(ct_ref, x_ref, mn_ref, mx_ref):
    i = pl.program_id(0)
    d = x_ref[...] * ct_ref[0, 0]
    bmn = jnp.min(d)
    bmx = jnp.max(d)

    @pl.when(i == 0)
    def _():
        mn_ref[0, 0] = bmn
        mx_ref[0, 0] = bmx

    @pl.when(i > 0)
    def _():
        mn_ref[0, 0] = jnp.minimum(mn_ref[0, 0], bmn)
        mx_ref[0, 0] = jnp.maximum(mx_ref[0, 0], bmx)


def _hist_kernel(ct_ref, mn_ref, mx_ref, x_ref, s_ref):
    i = pl.program_id(0)
    dmin = mn_ref[0, 0] - 0.1
    dmax = mx_ref[0, 0] + 0.1
    step = (dmax - dmin) / _P
    d = x_ref[...] * ct_ref[0, 0]

    @pl.when(i == 0)
    def _():
        for k in range(_P):
            s_ref[0, k] = 0

    for k in range(1, _P + 1):
        e = dmin + k * step
        s_ref[0, k - 1] += jnp.sum((d >= e).astype(jnp.int32))


def _finz(v):
    return jnp.where(jnp.abs(v) < jnp.inf, v, 0.0)


def _hinge3(x, a1, a2, a3, c, m, g1, g2, g3):
    return c + m * x + g1 * a1 + g2 * a2 + g3 * a3


def _pwl_coeffs(p, f32):
    s = [p[j + 1] - p[j] for j in range(4)]
    g1 = (s[1] - s[0]) * f32(0.5)
    g2 = (s[2] - s[1]) * f32(0.5)
    g3 = (s[3] - s[2]) * f32(0.5)
    m = s[0] + g1 + g2 + g3
    c = p[0] - (g1 + 2.0 * g2 + 3.0 * g3)
    return c, m, g1, g2, g3


def _map_kernel(nelem, ct_ref, st_ref, mn_ref, mx_ref, s_ref, prm_ref,
                x_ref, o_ref, pre_ref):
    f32 = jnp.float32

    @pl.when(pl.program_id(0) == 0)
    def _():
        _map_preamble(nelem, ct_ref, st_ref, mn_ref, mx_ref, s_ref, prm_ref,
                      pre_ref)

    v = [pre_ref[0, k] for k in range(20)]
    (ct, st, G1, G2, G3, cI, mI, gI1, gI2, gI3,
     cT, mT, gT1, gT2, gT3, cV, mV, gV1, gV2, gV3) = v

    d = x_ref[...] * ct
    a1 = jnp.abs(d - G1)
    a2 = jnp.abs(d - G2)
    a3 = jnp.abs(d - G3)
    index = _hinge3(d, a1, a2, a3, cI, mI, gI1, gI2, gI3)

    frac = index - jnp.floor(index)
    z = jnp.where(index < 0.0, frac, jnp.minimum(index, f32(4.0)))
    b1 = jnp.abs(z - f32(1.0))
    b2 = jnp.abs(z - f32(2.0))
    b3 = jnp.abs(z - f32(3.0))
    theta = _hinge3(z, b1, b2, b3, cT, mT, gT1, gT2, gT3)
    velo = _hinge3(z, b1, b2, b3, cV, mV, gV1, gV2, gV3)

    r = theta * f32(_INV2PI)
    u = r - jnp.round(r)
    T = u * u
    ps = f32(_SINC[6])
    pc = f32(_COSC[6])
    for k in range(5, -1, -1):
        ps = f32(_SINC[k]) + T * ps
        pc = f32(_COSC[k]) + T * pc
    sin_t = u * ps
    cos_t = pc

    dy = velo * sin_t
    dx = velo * cos_t
    o_ref[...] = (d * (1.0 + dy) + dx) * st


def _map_preamble(nelem, ct_ref, st_ref, mn_ref, mx_ref, s_ref, prm_ref,
                  pre_ref):
    f32 = jnp.float32
    ct = ct_ref[0, 0]
    st = st_ref[0, 0]
    dmin = mn_ref[0, 0] - 0.1
    dmax = mx_ref[0, 0] + 0.1
    step = (dmax - dmin) / _P
    e = [dmin + k * step for k in range(_P + 1)]

    S = [f32(nelem)] + [s_ref[0, k].astype(f32) for k in range(_P)]
    c = [S[k] - S[k + 1] for k in range(_P)]
    total = c[0] + c[1] + c[2] + c[3] + c[4]
    acc = []
    run = f32(0.0)
    for k in range(_P):
        run = run + c[k] / total
        acc.append(run * _P)
    grid = [(e[k] + e[k + 1]) * 0.5 for k in range(_P)]

    sl = [(acc[i + 1] - acc[i]) / (grid[i + 1] - grid[i]) for i in range(4)]
    gI1 = (sl[1] - sl[0]) * f32(0.5)
    gI2 = (sl[2] - sl[1]) * f32(0.5)
    gI3 = (sl[3] - sl[2]) * f32(0.5)
    mI = sl[0] + gI1 + gI2 + gI3
    cI = acc[1] - mI * grid[1] - gI2 * (grid[2] - grid[1]) \
        - gI3 * (grid[3] - grid[1])

    def sel4(idx, v):
        return jnp.where(idx == 0, v[0],
               jnp.where(idx == 1, v[1],
               jnp.where(idx == 2, v[2], v[3])))

    frame = []
    for j in range(_P):
        jf = f32(j)
        t = jnp.int32(0)
        for k in range(_P):
            t = t + jnp.where(acc[k] < jf, 1, 0)
        ij = jnp.clip(t - 1, 0, _P - 2)
        x0 = sel4(ij, acc[0:4])
        x1 = sel4(ij, acc[1:5])
        y0 = sel4(ij, grid[0:4])
        y1 = sel4(ij, grid[1:5])
        frame.append(y0 + (jf - x0) / (x1 - x0) * (y1 - y0))

    pt = [_finz(frame[k] + 0.001 * prm_ref[0, k]) for k in range(_P)]
    pv = [_finz(frame[k] + 0.001 * prm_ref[1, k]) for k in range(_P)]
    cT, mT, gT1, gT2, gT3 = _pwl_coeffs(pt, f32)
    cV, mV, gV1, gV2, gV3 = _pwl_coeffs(pv, f32)

    vals = (ct, st, grid[1], grid[2], grid[3], cI, mI, gI1, gI2, gI3,
            cT, mT, gT1, gT2, gT3, cV, mV, gV1, gV2, gV3)
    for k, val in enumerate(vals):
        pre_ref[0, k] = val


def _smem_spec():
    return pl.BlockSpec(memory_space=pltpu.SMEM)


@jax.jit
def kernel(data, params, channel_transform, spatio_transform):
    shape = data.shape
    cols = _BC
    rows = data.size // cols
    x = data.reshape(rows, cols)
    nblk = rows // _BR
    ct = channel_transform.reshape(1, 1)
    st = spatio_transform.reshape(1, 1)
    prm = params.reshape(2, _P)

    blk = pl.BlockSpec((_BR, cols), lambda i: (i, 0))

    mn, mx = pl.pallas_call(
        _minmax_kernel,
        grid=(nblk,),
        in_specs=[_smem_spec(), blk],
        out_specs=[_smem_spec(), _smem_spec()],
        out_shape=[jax.ShapeDtypeStruct((1, 1), jnp.float32),
                   jax.ShapeDtypeStruct((1, 1), jnp.float32)],
    )(ct, x)

    s = pl.pallas_call(
        _hist_kernel,
        grid=(nblk,),
        in_specs=[_smem_spec(), _smem_spec(), _smem_spec(), blk],
        out_specs=_smem_spec(),
        out_shape=jax.ShapeDtypeStruct((1, _P), jnp.int32),
    )(ct, mn, mx, x)

    out = pl.pallas_call(
        functools.partial(_map_kernel, float(rows * cols)),
        grid=(nblk,),
        in_specs=[_smem_spec()] * 6 + [blk],
        out_specs=blk,
        out_shape=jax.ShapeDtypeStruct((rows, cols), jnp.float32),
        scratch_shapes=[pltpu.SMEM((1, 20), jnp.float32)],
    )(ct, st, mn, mx, s, prm, x)

    return out.reshape(shape)
